# baseline (speedup 1.0000x reference)
import math

import jax
import jax.numpy as jnp
from jax.experimental import pallas as pl
from jax.experimental.pallas import tpu as pltpu

_K = 512
_L = 64
_B = 4096
_LANES = 128
_ROWS = _B * _L // _LANES
_RB = 8

_HALF_LOG_2PI = 0.5 * math.log(2.0 * math.pi)
_LOG2E = math.log2(math.e)
_LN2 = math.log(2.0)


def _coef_kernel(mt_ref, lvt_ref, w_ref, a_ref, b_ref, c_ref, m_ref):
    lv = lvt_ref[...]
    mu = mt_ref[...]
    wv = w_ref[...]
    wmax = jnp.max(wv, axis=0, keepdims=True)
    lse_w = wmax + jnp.log(jnp.sum(jnp.exp(wv - wmax), axis=0, keepdims=True))
    lw = wv - lse_w
    a2 = -0.5 * jnp.exp(-lv)
    c0 = _LOG2E * ((lw - _HALF_LOG_2PI) - 0.5 * lv)
    cap = jnp.max(c0, axis=0, keepdims=True)
    a_ref[...] = _LOG2E * a2
    b_ref[...] = _LOG2E * (-2.0 * a2) * mu
    c_ref[...] = (c0 - cap) + (_LOG2E * a2) * mu * mu
    m_ref[...] = cap


def _mog_kernel(z_ref, a_ref, b_ref, c_ref, m_ref, out_ref):
    alpha = a_ref[...]
    beta = b_ref[...]
    gamma = c_ref[...]
    cap = m_ref[...]
    for r in range(_RB):
        zrow = z_ref[r:r + 1, :]
        p = gamma + zrow * (beta + zrow * alpha)
        s = jnp.sum(jnp.exp2(p), axis=0, keepdims=True)
        s = jnp.maximum(s, 2.0 ** -140)
        out_ref[r:r + 1, :] = _LN2 * (cap + jnp.log2(s))


def kernel(z, means, logvars, w):
    z2 = z.reshape(_ROWS, _LANES)
    mt = jnp.concatenate([means, means], axis=1)
    lvt = jnp.concatenate([logvars, logvars], axis=1)
    wc = w.reshape(_K, 1)
    coef_shapes = (
        jax.ShapeDtypeStruct((_K, _LANES), jnp.float32),
        jax.ShapeDtypeStruct((_K, _LANES), jnp.float32),
        jax.ShapeDtypeStruct((_K, _LANES), jnp.float32),
        jax.ShapeDtypeStruct((1, _LANES), jnp.float32),
    )
    alpha, beta, gamma, cap = pl.pallas_call(
        _coef_kernel,
        out_shape=coef_shapes,
    )(mt, lvt, wc)
    out2 = pl.pallas_call(
        _mog_kernel,
        grid=(_ROWS // _RB,),
        in_specs=[
            pl.BlockSpec((_RB, _LANES), lambda i: (i, 0)),
            pl.BlockSpec((_K, _LANES), lambda i: (0, 0)),
            pl.BlockSpec((_K, _LANES), lambda i: (0, 0)),
            pl.BlockSpec((_K, _LANES), lambda i: (0, 0)),
            pl.BlockSpec((1, _LANES), lambda i: (0, 0)),
        ],
        out_specs=pl.BlockSpec((_RB, _LANES), lambda i: (i, 0)),
        out_shape=jax.ShapeDtypeStruct((_ROWS, _LANES), jnp.float32),
    )(z2, alpha, beta, gamma, cap)
    return out2.reshape(_B, _L)


# speedup vs baseline: 1.0010x; 1.0010x over previous
import math

import jax
import jax.numpy as jnp
from jax.experimental import pallas as pl
from jax.experimental.pallas import tpu as pltpu

_K = 512
_L = 64
_B = 4096
_LANES = 128
_ROWS = _B * _L // _LANES
_RB = 8

_HALF_LOG_2PI = 0.5 * math.log(2.0 * math.pi)
_LOG2E = math.log2(math.e)
_LN2 = math.log(2.0)


def _coef_kernel(mt_ref, lvt_ref, w_ref, a_ref, b_ref, c_ref, m_ref):
    lv = lvt_ref[...]
    mu = mt_ref[...]
    wv = w_ref[...]
    wmax = jnp.max(wv, axis=0, keepdims=True)
    lse_w = wmax + jnp.log(jnp.sum(jnp.exp(wv - wmax), axis=0, keepdims=True))
    lw = wv - lse_w
    a2 = -0.5 * jnp.exp(-lv)
    c0 = _LOG2E * ((lw - _HALF_LOG_2PI) - 0.5 * lv)
    cap = jnp.max(c0, axis=0, keepdims=True)
    a_ref[...] = _LOG2E * a2
    b_ref[...] = _LOG2E * (-2.0 * a2) * mu
    c_ref[...] = (c0 - cap) + (_LOG2E * a2) * mu * mu
    m_ref[...] = cap


def _mog_kernel(z_ref, a_ref, b_ref, c_ref, m_ref, out_ref):
    cap = m_ref[...]
    zrows = [z_ref[r:r + 1, :] for r in range(_RB)]
    accs = [None] * _RB
    for j in range(_K // 8):
        sl = slice(j * 8, (j + 1) * 8)
        aj = a_ref[sl, :]
        bj = b_ref[sl, :]
        cj = c_ref[sl, :]
        for r in range(_RB):
            zr = zrows[r]
            t = jnp.exp2(cj + zr * (bj + zr * aj))
            accs[r] = t if accs[r] is None else accs[r] + t
    for r in range(_RB):
        s = jnp.sum(accs[r], axis=0, keepdims=True)
        s = jnp.maximum(s, 2.0 ** -140)
        out_ref[r:r + 1, :] = _LN2 * (cap + jnp.log2(s))


def kernel(z, means, logvars, w):
    z2 = z.reshape(_ROWS, _LANES)
    mt = jnp.concatenate([means, means], axis=1)
    lvt = jnp.concatenate([logvars, logvars], axis=1)
    wc = w.reshape(_K, 1)
    coef_shapes = (
        jax.ShapeDtypeStruct((_K, _LANES), jnp.float32),
        jax.ShapeDtypeStruct((_K, _LANES), jnp.float32),
        jax.ShapeDtypeStruct((_K, _LANES), jnp.float32),
        jax.ShapeDtypeStruct((1, _LANES), jnp.float32),
    )
    alpha, beta, gamma, cap = pl.pallas_call(
        _coef_kernel,
        out_shape=coef_shapes,
    )(mt, lvt, wc)
    out2 = pl.pallas_call(
        _mog_kernel,
        grid=(_ROWS // _RB,),
        in_specs=[
            pl.BlockSpec((_RB, _LANES), lambda i: (i, 0)),
            pl.BlockSpec((_K, _LANES), lambda i: (0, 0)),
            pl.BlockSpec((_K, _LANES), lambda i: (0, 0)),
            pl.BlockSpec((_K, _LANES), lambda i: (0, 0)),
            pl.BlockSpec((1, _LANES), lambda i: (0, 0)),
        ],
        out_specs=pl.BlockSpec((_RB, _LANES), lambda i: (i, 0)),
        out_shape=jax.ShapeDtypeStruct((_ROWS, _LANES), jnp.float32),
    )(z2, alpha, beta, gamma, cap)
    return out2.reshape(_B, _L)


# speedup vs baseline: 1.3727x; 1.3712x over previous
import math

import jax
import jax.numpy as jnp
from jax.experimental import pallas as pl
from jax.experimental.pallas import tpu as pltpu

_K = 512
_L = 64
_B = 4096
_LANES = 128
_ROWS = _B * _L // _LANES
_RB = 32

_HALF_LOG_2PI = 0.5 * math.log(2.0 * math.pi)
_LOG2E = math.log2(math.e)
_LN2 = math.log(2.0)


def _coef_kernel(mt_ref, lvt_ref, w_ref, a_ref, b_ref, c_ref, m_ref):
    lv = lvt_ref[...]
    mu = mt_ref[...]
    wv = w_ref[...]
    wmax = jnp.max(wv, axis=0, keepdims=True)
    lse_w = wmax + jnp.log(jnp.sum(jnp.exp(wv - wmax), axis=0, keepdims=True))
    lw = wv - lse_w
    a2 = -0.5 * jnp.exp(-lv)
    c0 = _LOG2E * ((lw - _HALF_LOG_2PI) - 0.5 * lv)
    cap = jnp.max(c0, axis=0, keepdims=True)
    a_ref[...] = _LOG2E * a2
    b_ref[...] = _LOG2E * (-2.0 * a2) * mu
    c_ref[...] = (c0 - cap) + (_LOG2E * a2) * mu * mu
    m_ref[...] = cap


def _mog_kernel(z_ref, a_ref, b_ref, c_ref, m_ref, out_ref):
    cap = m_ref[...]
    for g in range(0, _RB, 8):
        zrows = [z_ref[g + r:g + r + 1, :] for r in range(8)]
        accs = [None] * 8
        for j in range(_K // 8):
            sl = slice(j * 8, (j + 1) * 8)
            aj = a_ref[sl, :]
            bj = b_ref[sl, :]
            cj = c_ref[sl, :]
            for r in range(8):
                zr = zrows[r]
                t = jnp.exp2(cj + zr * (bj + zr * aj))
                accs[r] = t if accs[r] is None else accs[r] + t
        for r in range(8):
            s = jnp.sum(accs[r], axis=0, keepdims=True)
            s = jnp.maximum(s, 2.0 ** -140)
            out_ref[g + r:g + r + 1, :] = _LN2 * (cap + jnp.log2(s))


def kernel(z, means, logvars, w):
    z2 = z.reshape(_ROWS, _LANES)
    mt = jnp.concatenate([means, means], axis=1)
    lvt = jnp.concatenate([logvars, logvars], axis=1)
    wc = w.reshape(_K, 1)
    coef_shapes = (
        jax.ShapeDtypeStruct((_K, _LANES), jnp.float32),
        jax.ShapeDtypeStruct((_K, _LANES), jnp.float32),
        jax.ShapeDtypeStruct((_K, _LANES), jnp.float32),
        jax.ShapeDtypeStruct((1, _LANES), jnp.float32),
    )
    alpha, beta, gamma, cap = pl.pallas_call(
        _coef_kernel,
        out_shape=coef_shapes,
    )(mt, lvt, wc)
    out2 = pl.pallas_call(
        _mog_kernel,
        grid=(_ROWS // _RB,),
        in_specs=[
            pl.BlockSpec((_RB, _LANES), lambda i: (i, 0)),
            pl.BlockSpec((_K, _LANES), lambda i: (0, 0)),
            pl.BlockSpec((_K, _LANES), lambda i: (0, 0)),
            pl.BlockSpec((_K, _LANES), lambda i: (0, 0)),
            pl.BlockSpec((1, _LANES), lambda i: (0, 0)),
        ],
        out_specs=pl.BlockSpec((_RB, _LANES), lambda i: (i, 0)),
        out_shape=jax.ShapeDtypeStruct((_ROWS, _LANES), jnp.float32),
    )(z2, alpha, beta, gamma, cap)
    return out2.reshape(_B, _L)


# speedup vs baseline: 1.3935x; 1.0152x over previous
import math

import jax
import jax.numpy as jnp
from jax.experimental import pallas as pl
from jax.experimental.pallas import tpu as pltpu

_K = 512
_L = 64
_B = 4096
_LANES = 128
_ROWS = _B * _L // _LANES
_RB = 64

_HALF_LOG_2PI = 0.5 * math.log(2.0 * math.pi)
_LOG2E = math.log2(math.e)
_LN2 = math.log(2.0)


def _coef_kernel(mt_ref, lvt_ref, w_ref, a_ref, b_ref, c_ref, m_ref):
    lv = lvt_ref[...]
    mu = mt_ref[...]
    wv = w_ref[...]
    wmax = jnp.max(wv, axis=0, keepdims=True)
    lse_w = wmax + jnp.log(jnp.sum(jnp.exp(wv - wmax), axis=0, keepdims=True))
    lw = wv - lse_w
    a2 = -0.5 * jnp.exp(-lv)
    c0 = _LOG2E * ((lw - _HALF_LOG_2PI) - 0.5 * lv)
    cap = jnp.max(c0, axis=0, keepdims=True)
    a_ref[...] = _LOG2E * a2
    b_ref[...] = _LOG2E * (-2.0 * a2) * mu
    c_ref[...] = (c0 - cap) + (_LOG2E * a2) * mu * mu
    m_ref[...] = cap


def _mog_kernel(z_ref, a_ref, b_ref, c_ref, m_ref, out_ref):
    cap = m_ref[...]
    for g in range(0, _RB, 8):
        zrows = [z_ref[g + r:g + r + 1, :] for r in range(8)]
        accs = [None] * 8
        for j in range(_K // 8):
            sl = slice(j * 8, (j + 1) * 8)
            aj = a_ref[sl, :]
            bj = b_ref[sl, :]
            cj = c_ref[sl, :]
            for r in range(8):
                zr = zrows[r]
                t = jnp.exp2(cj + zr * (bj + zr * aj))
                accs[r] = t if accs[r] is None else accs[r] + t
        for r in range(8):
            s = jnp.sum(accs[r], axis=0, keepdims=True)
            s = jnp.maximum(s, 2.0 ** -140)
            out_ref[g + r:g + r + 1, :] = _LN2 * (cap + jnp.log2(s))


def kernel(z, means, logvars, w):
    z2 = z.reshape(_ROWS, _LANES)
    mt = jnp.concatenate([means, means], axis=1)
    lvt = jnp.concatenate([logvars, logvars], axis=1)
    wc = w.reshape(_K, 1)
    coef_shapes = (
        jax.ShapeDtypeStruct((_K, _LANES), jnp.float32),
        jax.ShapeDtypeStruct((_K, _LANES), jnp.float32),
        jax.ShapeDtypeStruct((_K, _LANES), jnp.float32),
        jax.ShapeDtypeStruct((1, _LANES), jnp.float32),
    )
    alpha, beta, gamma, cap = pl.pallas_call(
        _coef_kernel,
        out_shape=coef_shapes,
    )(mt, lvt, wc)
    out2 = pl.pallas_call(
        _mog_kernel,
        grid=(_ROWS // _RB,),
        in_specs=[
            pl.BlockSpec((_RB, _LANES), lambda i: (i, 0)),
            pl.BlockSpec((_K, _LANES), lambda i: (0, 0)),
            pl.BlockSpec((_K, _LANES), lambda i: (0, 0)),
            pl.BlockSpec((_K, _LANES), lambda i: (0, 0)),
            pl.BlockSpec((1, _LANES), lambda i: (0, 0)),
        ],
        out_specs=pl.BlockSpec((_RB, _LANES), lambda i: (i, 0)),
        out_shape=jax.ShapeDtypeStruct((_ROWS, _LANES), jnp.float32),
    )(z2, alpha, beta, gamma, cap)
    return out2.reshape(_B, _L)
